# baseline (speedup 1.0000x reference)
import jax
import jax.numpy as jnp
from jax import lax
from jax.experimental import pallas as pl
from jax.experimental.pallas import tpu as pltpu
from jax.experimental.pallas import tpu_sc as plsc

N = 50000
D = 256
G = 128
NC = 2
NS = 16
NW = NC * NS
C = 1568
K = 112
NCH = C // K
NV = D // 16


def _seg_sum_body(feat_hbm, ids_hbm, part_hbm, pcnt_hbm,
                  buf0, buf1, idv0, idv1, accv, cntv, curv, ccur, prevs,
                  sem0, sem1):
    c = lax.axis_index("c")
    s = lax.axis_index("s")
    w = s * NC + c
    r0 = w * C
    r1 = jnp.minimum(r0 + C, N)

    bufs = (buf0, buf1)
    idvs = (idv0, idv1)
    sems = (sem0, sem1)

    zeros16 = jnp.zeros((16,), jnp.float32)
    ones16 = jnp.ones((16,), jnp.float32)
    zi16 = jnp.zeros((16,), jnp.int32)
    iota16 = lax.iota(jnp.int32, 16)
    lanemasks = [iota16 == jnp.full((16,), j, jnp.int32) for j in range(16)]

    def chunk_start(ci):
        p0 = r0 + ci * K
        p1 = jnp.minimum(p0 + K, r1)
        sdma = jnp.minimum(p0, N - K)
        return p0, p1, sdma

    def zbody(g, carry):
        cntv[g, :] = zeros16
        return carry

    lax.fori_loop(0, G, zbody, 0)
    for k in range(NV):
        curv[pl.ds(k * 16, 16)] = zeros16
    ccur[:] = zeros16
    prevs[0] = jnp.int32(-1)

    _, _, sd0 = chunk_start(0)
    pltpu.async_copy(feat_hbm.at[pl.ds(sd0, K)], buf0, sem0)
    pltpu.async_copy(ids_hbm.at[pl.ds(sd0, K)], idv0, sem0)

    def outer_body(o, carry):
        for b in range(2):
            ci = 2 * o + b
            p0, p1, sdma = chunk_start(ci)
            buf, idv, sem = bufs[b], idvs[b], sems[b]
            pltpu.make_async_copy(feat_hbm.at[pl.ds(sdma, K)], buf, sem).wait()
            pltpu.make_async_copy(ids_hbm.at[pl.ds(sdma, K)], idv, sem).wait()

            @pl.when(ci + 1 < NCH)
            def _():
                _, _, sdn = chunk_start(ci + 1)
                nb = 1 - b
                pltpu.async_copy(feat_hbm.at[pl.ds(sdn, K)], bufs[nb],
                                 sems[nb])
                pltpu.async_copy(ids_hbm.at[pl.ds(sdn, K)], idvs[nb],
                                 sems[nb])

            full = (sdma == p0) & (p1 == p0 + K)

            @pl.when(full)
            def _():
                def group_body(t, carry):
                    prev, cntf, acc = carry
                    idvec = idv[pl.ds(t * 16, 16)]
                    for j in range(16):
                        r = t * 16 + j
                        g = jnp.sum(jnp.where(lanemasks[j], idvec, zi16))
                        same = (g == prev).astype(jnp.float32)
                        samev = jnp.full((16,), same, jnp.float32)
                        acc = tuple(
                            acc[k] * samev + buf[r, pl.ds(k * 16, 16)]
                            for k in range(NV))
                        cntf = cntf * samev + ones16
                        for k in range(NV):
                            accv[g, pl.ds(k * 16, 16)] = acc[k]
                        cntv[g, :] = cntf
                        prev = g
                    return prev, cntf, acc

                init = (prevs[0], ccur[:],
                        tuple(curv[pl.ds(k * 16, 16)] for k in range(NV)))
                prev, cntf, acc = lax.fori_loop(0, K // 16, group_body, init)
                prevs[0] = prev
                ccur[:] = cntf
                for k in range(NV):
                    curv[pl.ds(k * 16, 16)] = acc[k]

            @pl.when(jnp.logical_not(full))
            def _():
                def group_body(t, carry):
                    prev, cntf, acc = carry
                    idvec = idv[pl.ds(t * 16, 16)]
                    for j in range(16):
                        r = t * 16 + j
                        g = jnp.sum(jnp.where(lanemasks[j], idvec, zi16))
                        rowi = sdma + r
                        inr = (rowi >= p0) & (rowi < p1)
                        inrf = inr.astype(jnp.float32)
                        same = (g == prev).astype(jnp.float32)
                        keep = jnp.maximum(same, 1.0 - inrf)
                        keepv = jnp.full((16,), keep, jnp.float32)
                        inrv = jnp.full((16,), inrf, jnp.float32)
                        acc = tuple(
                            acc[k] * keepv +
                            buf[r, pl.ds(k * 16, 16)] * inrv
                            for k in range(NV))
                        cntf = cntf * keepv + inrv
                        acc_now = acc
                        cnt_now = cntf

                        @pl.when(inr)
                        def _():
                            for k in range(NV):
                                accv[g, pl.ds(k * 16, 16)] = acc_now[k]
                            cntv[g, :] = cnt_now

                        prev = g
                    return prev, cntf, acc

                init = (prevs[0], ccur[:],
                        tuple(curv[pl.ds(k * 16, 16)] for k in range(NV)))
                prev, cntf, acc = lax.fori_loop(0, K // 16, group_body, init)
                prevs[0] = prev
                ccur[:] = cntf
                for k in range(NV):
                    curv[pl.ds(k * 16, 16)] = acc[k]

        return carry

    lax.fori_loop(0, NCH // 2, outer_body, 0)

    pltpu.sync_copy(accv, part_hbm.at[w])
    pltpu.sync_copy(cntv, pcnt_hbm.at[w])


def _seg_sum(feat, ids):
    mesh = plsc.VectorSubcoreMesh(core_axis_name="c", subcore_axis_name="s",
                                  num_cores=NC, num_subcores=NS)
    fn = pl.kernel(
        _seg_sum_body,
        out_type=(jax.ShapeDtypeStruct((NW, G, D), jnp.float32),
                  jax.ShapeDtypeStruct((NW, G, 16), jnp.float32)),
        mesh=mesh,
        compiler_params=pltpu.CompilerParams(needs_layout_passes=False),
        scratch_types=[
            pltpu.VMEM((K, D), jnp.float32),
            pltpu.VMEM((K, D), jnp.float32),
            pltpu.VMEM((K,), jnp.int32),
            pltpu.VMEM((K,), jnp.int32),
            pltpu.VMEM((G, D), jnp.float32),
            pltpu.VMEM((G, 16), jnp.float32),
            pltpu.VMEM((D,), jnp.float32),
            pltpu.VMEM((16,), jnp.float32),
            pltpu.SMEM((8,), jnp.int32),
            pltpu.SemaphoreType.DMA,
            pltpu.SemaphoreType.DMA,
        ],
    )
    return fn(feat, ids)


def _finalize_body(part_ref, pc_ref, out_ref):
    def body(wi, a):
        touched = pc_ref[wi, :, 0:1] > 0.0
        return a + jnp.where(touched, part_ref[wi], 0.0)

    acc = lax.fori_loop(0, NW, body, jnp.zeros((G, D), jnp.float32))

    def body2(wi, a):
        return a + pc_ref[wi]

    cnt = lax.fori_loop(0, NW, body2, jnp.zeros((G, 16), jnp.float32))
    c1 = jnp.maximum(cnt[:, 0:1], 1.0)
    out_ref[...] = acc / c1


def _finalize(partials, pcnt):
    return pl.pallas_call(
        _finalize_body,
        out_shape=jax.ShapeDtypeStruct((G, D), jnp.float32),
    )(partials, pcnt)


@jax.jit
def kernel(feat, graph_ids):
    ids = graph_ids.astype(jnp.int32)
    partials, pcnt = _seg_sum(feat, ids)
    return _finalize(partials, pcnt)


# speedup vs baseline: 1.0503x; 1.0503x over previous
import jax
import jax.numpy as jnp
from jax import lax
from jax.experimental import pallas as pl
from jax.experimental.pallas import tpu as pltpu
from jax.experimental.pallas import tpu_sc as plsc

N = 50000
D = 256
G = 128
NC = 2
NS = 16
NW = NC * NS
C = 1568
K = 112
NCH = C // K
NV = D // 16


def _seg_sum_body(feat_hbm, ids_hbm, part_hbm, pcnt_hbm,
                  buf0, buf1, idv0, idv1, accv, cntv, curv, ccur, prevs,
                  sem0, sem1):
    c = lax.axis_index("c")
    s = lax.axis_index("s")
    w = s * NC + c
    r0 = w * C
    r1 = jnp.minimum(r0 + C, N)

    bufs = (buf0, buf1)
    idvs = (idv0, idv1)
    sems = (sem0, sem1)

    zeros16 = jnp.zeros((16,), jnp.float32)
    ones16 = jnp.ones((16,), jnp.float32)
    zi16 = jnp.zeros((16,), jnp.int32)
    iota16 = lax.iota(jnp.int32, 16)
    lanemasks = [iota16 == jnp.full((16,), j, jnp.int32) for j in range(16)]

    def chunk_start(ci):
        p0 = r0 + ci * K
        p1 = jnp.minimum(p0 + K, r1)
        sdma = jnp.minimum(p0, N - K)
        return p0, p1, sdma

    def zbody(g, carry):
        for k in range(NV):
            accv[g, pl.ds(k * 16, 16)] = zeros16
        cntv[g, :] = zeros16
        return carry

    lax.fori_loop(0, G, zbody, 0)
    for k in range(NV):
        curv[pl.ds(k * 16, 16)] = zeros16
    ccur[:] = zeros16
    prevs[0] = jnp.int32(-1)

    _, _, sd0 = chunk_start(0)
    pltpu.async_copy(feat_hbm.at[pl.ds(sd0, K)], buf0, sem0)
    pltpu.async_copy(ids_hbm.at[pl.ds(sd0, K)], idv0, sem0)

    def outer_body(o, carry):
        for b in range(2):
            ci = 2 * o + b
            p0, p1, sdma = chunk_start(ci)
            buf, idv, sem = bufs[b], idvs[b], sems[b]
            pltpu.make_async_copy(feat_hbm.at[pl.ds(sdma, K)], buf, sem).wait()
            pltpu.make_async_copy(ids_hbm.at[pl.ds(sdma, K)], idv, sem).wait()

            @pl.when(ci + 1 < NCH)
            def _():
                _, _, sdn = chunk_start(ci + 1)
                nb = 1 - b
                pltpu.async_copy(feat_hbm.at[pl.ds(sdn, K)], bufs[nb],
                                 sems[nb])
                pltpu.async_copy(ids_hbm.at[pl.ds(sdn, K)], idvs[nb],
                                 sems[nb])

            full = (sdma == p0) & (p1 == p0 + K)

            @pl.when(full)
            def _():
                def group_body(t, carry):
                    prev, cntf, acc = carry
                    idvec = idv[pl.ds(t * 16, 16)]
                    for j in range(16):
                        r = t * 16 + j
                        g = jnp.sum(jnp.where(lanemasks[j], idvec, zi16))
                        same = (g == prev).astype(jnp.float32)
                        samev = jnp.full((16,), same, jnp.float32)
                        acc = tuple(
                            acc[k] * samev + buf[r, pl.ds(k * 16, 16)]
                            for k in range(NV))
                        cntf = cntf * samev + ones16
                        for k in range(NV):
                            accv[g, pl.ds(k * 16, 16)] = acc[k]
                        cntv[g, :] = cntf
                        prev = g
                    return prev, cntf, acc

                init = (prevs[0], ccur[:],
                        tuple(curv[pl.ds(k * 16, 16)] for k in range(NV)))
                prev, cntf, acc = lax.fori_loop(0, K // 16, group_body, init)
                prevs[0] = prev
                ccur[:] = cntf
                for k in range(NV):
                    curv[pl.ds(k * 16, 16)] = acc[k]

            @pl.when(jnp.logical_not(full))
            def _():
                def group_body(t, carry):
                    prev, cntf, acc = carry
                    idvec = idv[pl.ds(t * 16, 16)]
                    for j in range(16):
                        r = t * 16 + j
                        g = jnp.sum(jnp.where(lanemasks[j], idvec, zi16))
                        rowi = sdma + r
                        inr = (rowi >= p0) & (rowi < p1)
                        inrf = inr.astype(jnp.float32)
                        same = (g == prev).astype(jnp.float32)
                        keep = jnp.maximum(same, 1.0 - inrf)
                        keepv = jnp.full((16,), keep, jnp.float32)
                        inrv = jnp.full((16,), inrf, jnp.float32)
                        acc = tuple(
                            acc[k] * keepv +
                            buf[r, pl.ds(k * 16, 16)] * inrv
                            for k in range(NV))
                        cntf = cntf * keepv + inrv
                        acc_now = acc
                        cnt_now = cntf

                        @pl.when(inr)
                        def _():
                            for k in range(NV):
                                accv[g, pl.ds(k * 16, 16)] = acc_now[k]
                            cntv[g, :] = cnt_now

                        prev = g
                    return prev, cntf, acc

                init = (prevs[0], ccur[:],
                        tuple(curv[pl.ds(k * 16, 16)] for k in range(NV)))
                prev, cntf, acc = lax.fori_loop(0, K // 16, group_body, init)
                prevs[0] = prev
                ccur[:] = cntf
                for k in range(NV):
                    curv[pl.ds(k * 16, 16)] = acc[k]

        return carry

    lax.fori_loop(0, NCH // 2, outer_body, 0)

    pltpu.sync_copy(accv, part_hbm.at[w])
    pltpu.sync_copy(cntv, pcnt_hbm.at[w])


def _seg_sum(feat, ids):
    mesh = plsc.VectorSubcoreMesh(core_axis_name="c", subcore_axis_name="s",
                                  num_cores=NC, num_subcores=NS)
    fn = pl.kernel(
        _seg_sum_body,
        out_type=(jax.ShapeDtypeStruct((NW, G, D), jnp.float32),
                  jax.ShapeDtypeStruct((NW, G, 16), jnp.float32)),
        mesh=mesh,
        compiler_params=pltpu.CompilerParams(needs_layout_passes=False),
        scratch_types=[
            pltpu.VMEM((K, D), jnp.float32),
            pltpu.VMEM((K, D), jnp.float32),
            pltpu.VMEM((K,), jnp.int32),
            pltpu.VMEM((K,), jnp.int32),
            pltpu.VMEM((G, D), jnp.float32),
            pltpu.VMEM((G, 16), jnp.float32),
            pltpu.VMEM((D,), jnp.float32),
            pltpu.VMEM((16,), jnp.float32),
            pltpu.SMEM((8,), jnp.int32),
            pltpu.SemaphoreType.DMA,
            pltpu.SemaphoreType.DMA,
        ],
    )
    return fn(feat, ids)


def _finalize_body(part_ref, pc_ref, out_ref):
    def body(wi, a):
        return a + part_ref[wi]

    acc = lax.fori_loop(0, NW, body, jnp.zeros((G, D), jnp.float32))

    def body2(wi, a):
        return a + pc_ref[wi]

    cnt = lax.fori_loop(0, NW, body2, jnp.zeros((G, 16), jnp.float32))
    c1 = jnp.maximum(cnt[:, 0:1], 1.0)
    out_ref[...] = acc / c1


def _finalize(partials, pcnt):
    return pl.pallas_call(
        _finalize_body,
        out_shape=jax.ShapeDtypeStruct((G, D), jnp.float32),
    )(partials, pcnt)


@jax.jit
def kernel(feat, graph_ids):
    ids = graph_ids.astype(jnp.int32)
    partials, pcnt = _seg_sum(feat, ids)
    return _finalize(partials, pcnt)
